# baseline (speedup 1.0000x reference)
import functools

import jax
import jax.numpy as jnp
from jax import lax
from jax.experimental import pallas as pl
from jax.experimental.pallas import tpu as pltpu
from jax.experimental.pallas import tpu_sc as plsc

_L = 16
_NC = 2
_NS = 16
_NW = _NC * _NS
_TBL = 2048
_B, _S, _D = 2, 8192, 2048


def _make_sc_kernel(rows_per_chunk: int, unroll: int):
  nrows = _B * _S
  assert nrows % _NW == 0
  rw = nrows // _NW
  nch = rw // rows_per_chunk
  assert nch % 2 == 0
  mesh = plsc.VectorSubcoreMesh(core_axis_name="c", subcore_axis_name="s")

  @functools.partial(
      pl.kernel,
      mesh=mesh,
      compiler_params=pltpu.CompilerParams(needs_layout_passes=False),
      out_type=jax.ShapeDtypeStruct((_B, _S, _D), jnp.float32),
      scratch_types=[
          pltpu.VMEM((_TBL,), jnp.float32),
          pltpu.VMEM((_L,), jnp.float32),
          pltpu.VMEM((rows_per_chunk, _D), jnp.int32),
          pltpu.VMEM((rows_per_chunk, _D), jnp.int32),
          pltpu.VMEM((rows_per_chunk, _D), jnp.float32),
          pltpu.VMEM((rows_per_chunk, _D), jnp.float32),
          pltpu.SemaphoreType.DMA,
          pltpu.SemaphoreType.DMA,
          pltpu.SemaphoreType.DMA,
          pltpu.SemaphoreType.DMA,
      ],
  )
  def body(inputs_3d, table_hbm, scales_hbm, out_3d,
           tab, scl, idx0, idx1, val0, val1, si0, si1, so0, so1):
    inputs_hbm = inputs_3d.reshape(nrows, _D)
    out_hbm = out_3d.reshape(nrows, _D)
    wid = lax.axis_index("s") * _NC + lax.axis_index("c")
    row_base = wid * rw

    pltpu.sync_copy(table_hbm, tab)
    pltpu.sync_copy(scales_hbm, scl)
    s = scl[...]

    @plsc.parallel_loop(0, _TBL, step=_L)
    def _(i):
      tab[pl.ds(i, _L)] = tab[pl.ds(i, _L)] * s

    idx_bufs = (idx0, idx1)
    val_bufs = (val0, val1)
    sin = (si0, si1)
    sout = (so0, so1)

    def in_copy(g, b):
      return pltpu.make_async_copy(
          inputs_hbm.at[pl.ds(row_base + g * rows_per_chunk, rows_per_chunk)],
          idx_bufs[b], sin[b])

    def out_copy(g, b):
      return pltpu.make_async_copy(
          val_bufs[b],
          out_hbm.at[pl.ds(row_base + g * rows_per_chunk, rows_per_chunk)],
          sout[b])

    in_copy(0, 0).start()
    in_copy(1, 1).start()

    def pair(p, carry):
      for b in range(2):
        g = p * 2 + b
        in_copy(g, b).wait()

        @pl.when(p > 0)
        def _():
          out_copy(g - 2, b).wait()

        idx_ref = idx_bufs[b]
        val_ref = val_bufs[b]

        for r in range(rows_per_chunk):
          @plsc.parallel_loop(0, _D, step=_L, unroll=unroll)
          def _(i, r=r):
            val_ref[r, pl.ds(i, _L)] = plsc.load_gather(
                tab, [idx_ref[r, pl.ds(i, _L)]])

        out_copy(g, b).start()

        @pl.when(p + 1 < nch // 2)
        def _():
          in_copy(g + 2, b).start()
      return carry

    lax.fori_loop(0, nch // 2, pair, 0)
    out_copy(nch - 2, 0).wait()
    out_copy(nch - 1, 1).wait()

  return body


_sc_kernel = _make_sc_kernel(rows_per_chunk=8, unroll=16)


@jax.jit
def kernel(inputs, table, scales):
  scl16 = jnp.broadcast_to(jnp.reshape(scales, (1,)), (_L,))
  return _sc_kernel(inputs, table, scl16)


# speedup vs baseline: 1.0864x; 1.0864x over previous
import functools

import jax
import jax.numpy as jnp
from jax import lax
from jax.experimental import pallas as pl
from jax.experimental.pallas import tpu as pltpu
from jax.experimental.pallas import tpu_sc as plsc

_L = 16
_NC = 2
_NS = 16
_NW = _NC * _NS
_TBL = 2048
_B, _S, _D = 2, 8192, 2048


def _make_sc_kernel(rows_per_chunk: int, unroll: int, nslots: int):
  nrows = _B * _S
  assert nrows % _NW == 0
  rw = nrows // _NW
  nch = rw // rows_per_chunk
  assert nch % nslots == 0 and nch // nslots >= 2
  mesh = plsc.VectorSubcoreMesh(core_axis_name="c", subcore_axis_name="s")

  scratch = [
      pltpu.VMEM((_TBL,), jnp.float32),
      pltpu.VMEM((_L,), jnp.float32),
  ]
  scratch += [pltpu.VMEM((rows_per_chunk, _D), jnp.int32)
              for _ in range(nslots)]
  scratch += [pltpu.VMEM((rows_per_chunk, _D), jnp.float32)
              for _ in range(nslots)]
  scratch += [pltpu.SemaphoreType.DMA for _ in range(2 * nslots)]

  @functools.partial(
      pl.kernel,
      mesh=mesh,
      compiler_params=pltpu.CompilerParams(needs_layout_passes=False),
      out_type=jax.ShapeDtypeStruct((_B, _S, _D), jnp.float32),
      scratch_types=scratch,
  )
  def body(inputs_3d, table_hbm, scales_hbm, out_3d, tab, scl, *bufs):
    idx_bufs = bufs[:nslots]
    val_bufs = bufs[nslots:2 * nslots]
    sin = bufs[2 * nslots:3 * nslots]
    sout = bufs[3 * nslots:]

    inputs_hbm = inputs_3d.reshape(nrows, _D)
    out_hbm = out_3d.reshape(nrows, _D)
    wid = lax.axis_index("s") * _NC + lax.axis_index("c")
    row_base = wid * rw

    pltpu.sync_copy(table_hbm, tab)
    pltpu.sync_copy(scales_hbm, scl)
    s = scl[...]

    @plsc.parallel_loop(0, _TBL, step=_L)
    def _(i):
      tab[pl.ds(i, _L)] = tab[pl.ds(i, _L)] * s

    def in_copy(g, b):
      return pltpu.make_async_copy(
          inputs_hbm.at[pl.ds(row_base + g * rows_per_chunk, rows_per_chunk)],
          idx_bufs[b], sin[b])

    def out_copy(g, b):
      return pltpu.make_async_copy(
          val_bufs[b],
          out_hbm.at[pl.ds(row_base + g * rows_per_chunk, rows_per_chunk)],
          sout[b])

    for b in range(nslots):
      in_copy(b, b).start()

    def step(p, carry):
      for b in range(nslots):
        g = p * nslots + b
        in_copy(g, b).wait()

        @pl.when(p > 0)
        def _():
          out_copy(g - nslots, b).wait()

        idx_ref = idx_bufs[b]
        val_ref = val_bufs[b]

        for r in range(rows_per_chunk):
          @plsc.parallel_loop(0, _D, step=_L, unroll=unroll)
          def _(i, r=r):
            val_ref[r, pl.ds(i, _L)] = plsc.load_gather(
                tab, [idx_ref[r, pl.ds(i, _L)]])

        out_copy(g, b).start()

        @pl.when(p + 1 < nch // nslots)
        def _():
          in_copy(g + nslots, b).start()
      return carry

    lax.fori_loop(0, nch // nslots, step, 0)
    for b in range(nslots):
      out_copy(nch - nslots + b, b).wait()

  return body


_sc_kernel = _make_sc_kernel(rows_per_chunk=4, unroll=8, nslots=4)


@jax.jit
def kernel(inputs, table, scales):
  scl16 = jnp.broadcast_to(jnp.reshape(scales, (1,)), (_L,))
  return _sc_kernel(inputs, table, scl16)


# speedup vs baseline: 1.0871x; 1.0007x over previous
import functools

import jax
import jax.numpy as jnp
from jax import lax
from jax.experimental import pallas as pl
from jax.experimental.pallas import tpu as pltpu
from jax.experimental.pallas import tpu_sc as plsc

_L = 16
_NC = 2
_NS = 16
_NW = _NC * _NS
_TBL = 2048
_B, _S, _D = 2, 8192, 2048


def _make_sc_kernel(rows_per_chunk: int, unroll: int, nslots: int):
  nrows = _B * _S
  assert nrows % _NW == 0
  rw = nrows // _NW
  nch = rw // rows_per_chunk
  assert nch % nslots == 0 and nch // nslots >= 2
  mesh = plsc.VectorSubcoreMesh(core_axis_name="c", subcore_axis_name="s")

  scratch = [
      pltpu.VMEM((_TBL,), jnp.float32),
      pltpu.VMEM((_L,), jnp.float32),
  ]
  scratch += [pltpu.VMEM((rows_per_chunk, _D), jnp.int32)
              for _ in range(nslots)]
  scratch += [pltpu.VMEM((rows_per_chunk, _D), jnp.float32)
              for _ in range(nslots)]
  scratch += [pltpu.SemaphoreType.DMA for _ in range(2 * nslots)]

  @functools.partial(
      pl.kernel,
      mesh=mesh,
      compiler_params=pltpu.CompilerParams(needs_layout_passes=False),
      out_type=jax.ShapeDtypeStruct((_B, _S, _D), jnp.float32),
      scratch_types=scratch,
  )
  def body(inputs_3d, table_hbm, scales_hbm, out_3d, tab, scl, *bufs):
    idx_bufs = bufs[:nslots]
    val_bufs = bufs[nslots:2 * nslots]
    sin = bufs[2 * nslots:3 * nslots]
    sout = bufs[3 * nslots:]

    inputs_hbm = inputs_3d.reshape(nrows, _D)
    out_hbm = out_3d.reshape(nrows, _D)
    wid = lax.axis_index("s") * _NC + lax.axis_index("c")
    row_base = wid * rw

    pltpu.sync_copy(table_hbm, tab)
    pltpu.sync_copy(scales_hbm, scl)
    s = scl[...]

    @plsc.parallel_loop(0, _TBL, step=_L)
    def _(i):
      tab[pl.ds(i, _L)] = tab[pl.ds(i, _L)] * s

    def in_copy(g, b):
      return pltpu.make_async_copy(
          inputs_hbm.at[pl.ds(row_base + g * rows_per_chunk, rows_per_chunk)],
          idx_bufs[b], sin[b])

    def out_copy(g, b):
      return pltpu.make_async_copy(
          val_bufs[b],
          out_hbm.at[pl.ds(row_base + g * rows_per_chunk, rows_per_chunk)],
          sout[b])

    for b in range(nslots):
      in_copy(b, b).start()

    def step(p, carry):
      for b in range(nslots):
        g = p * nslots + b
        in_copy(g, b).wait()

        @pl.when(p > 0)
        def _():
          out_copy(g - nslots, b).wait()

        idx_ref = idx_bufs[b]
        val_ref = val_bufs[b]

        for r in range(rows_per_chunk):
          @plsc.parallel_loop(0, _D, step=_L, unroll=unroll)
          def _(i, r=r):
            val_ref[r, pl.ds(i, _L)] = plsc.load_gather(
                tab, [idx_ref[r, pl.ds(i, _L)]])

        out_copy(g, b).start()

        @pl.when(p + 1 < nch // nslots)
        def _():
          in_copy(g + nslots, b).start()
      return carry

    lax.fori_loop(0, nch // nslots, step, 0)
    for b in range(nslots):
      out_copy(nch - nslots + b, b).wait()

  return body


_sc_kernel = _make_sc_kernel(rows_per_chunk=4, unroll=16, nslots=4)


@jax.jit
def kernel(inputs, table, scales):
  scl16 = jnp.broadcast_to(jnp.reshape(scales, (1,)), (_L,))
  return _sc_kernel(inputs, table, scl16)
